# Initial kernel scaffold; baseline (speedup 1.0000x reference)
import jax
import jax.numpy as jnp
from jax.experimental import pallas as pl


def kernel(features, img_size, conv_w, conv_b, cls_w, cls_b, reg_w, reg_b):
    raise NotImplementedError("write your pallas kernel here")


# speedup vs baseline: 1.0025x; 1.0025x over previous
import functools
import jax
import jax.numpy as jnp
import numpy as np
from jax import lax
from jax.experimental import pallas as pl
from jax.experimental.pallas import tpu as pltpu

B, C, H, W = 4, 256, 50, 50
K = 9
PRE_K = 2000
POST_K = 1000
NMS_THR = 0.7
IMG = 800
STRIDE = IMG // H
WP = W + 2
PADG = WP * WP
XROWS = 2816


def _make_anchors_k():
    scales = [128.0, 256.0, 512.0]
    ratios = [0.5, 1.0, 2.0]
    ws, hs = [], []
    for s in scales:
        for r in ratios:
            ws.append(s * np.sqrt(1.0 / r))
            hs.append(s * np.sqrt(r))
    ws = np.array(ws, np.float32)
    hs = np.array(hs, np.float32)
    cx = (np.arange(W, dtype=np.float32) + 0.5) * STRIDE
    cy = (np.arange(H, dtype=np.float32) + 0.5) * STRIDE
    cxg, cyg = np.meshgrid(cx, cy)
    anch = np.zeros((H, W, K, 4), np.float32)
    anch[..., 0] = cxg[..., None]
    anch[..., 1] = cyg[..., None]
    anch[..., 2] = ws
    anch[..., 3] = hs
    return jnp.asarray(anch.reshape(-1, 4))


_ANCH = _make_anchors_k()


def _head_kernel(xpad_ref, w9_ref, cb_ref, hw_ref, hb_ref, obj_ref, dlt_ref, acc):
    for k in range(9):
        dh, dw = k // 3, k % 3
        off = dh * WP + dw
        part = jnp.dot(xpad_ref[0, off:off + PADG, :], w9_ref[k],
                       preferred_element_type=jnp.float32)
        if k == 0:
            acc[...] = part
        else:
            acc[...] += part
    x1 = jnp.maximum(acc[...] + cb_ref[0][None, :], 0.0)
    head = jnp.dot(x1, hw_ref[...], preferred_element_type=jnp.float32)
    head = head + hb_ref[0][None, :]
    obj = jax.nn.sigmoid(head[:, 0:K])
    obj_ref[0] = jnp.pad(obj, ((0, 0), (0, 16 - K)))
    dlt_ref[0] = head


def _rpn_head(features, conv_w, conv_b, cls_w, cls_b, reg_w, reg_b):
    xn = jnp.transpose(features, (0, 2, 3, 1))
    xp = jnp.pad(xn, ((0, 0), (1, 1), (1, 1), (0, 0)))
    xp = xp.reshape(B, PADG, C)
    xp = jnp.pad(xp, ((0, 0), (0, XROWS - PADG), (0, 0)))
    w9 = jnp.transpose(conv_w, (2, 3, 1, 0)).reshape(9, C, C)
    hw = jnp.concatenate([
        jnp.transpose(cls_w.reshape(K, C), (1, 0)),
        jnp.transpose(reg_w.reshape(4 * K, C), (1, 0)),
        jnp.zeros((C, 3), jnp.float32)], axis=1)
    hb = jnp.concatenate([cls_b, reg_b, jnp.zeros((3,), jnp.float32)])

    obj, dlt = pl.pallas_call(
        _head_kernel,
        grid=(B,),
        in_specs=[
            pl.BlockSpec((1, XROWS, C), lambda b: (b, 0, 0)),
            pl.BlockSpec((9, C, C), lambda b: (0, 0, 0)),
            pl.BlockSpec((1, C), lambda b: (0, 0)),
            pl.BlockSpec((C, 48), lambda b: (0, 0)),
            pl.BlockSpec((1, 48), lambda b: (0, 0)),
        ],
        out_specs=[
            pl.BlockSpec((1, PADG, 16), lambda b: (b, 0, 0)),
            pl.BlockSpec((1, PADG, 48), lambda b: (b, 0, 0)),
        ],
        out_shape=[
            jax.ShapeDtypeStruct((B, PADG, 16), jnp.float32),
            jax.ShapeDtypeStruct((B, PADG, 48), jnp.float32),
        ],
        scratch_shapes=[pltpu.VMEM((PADG, C), jnp.float32)],
    )(xp, w9, conv_b[None, :], hw, hb[None, :])

    obj = obj.reshape(B, WP, WP, 16)[:, :H, :W, :K]
    dlt = dlt.reshape(B, WP, WP, 48)[:, :H, :W, K:K + 4 * K]
    scores = obj.reshape(B, H * W * K)
    deltas = dlt.reshape(B, H * W, K, 4).reshape(B, H * W * K, 4)
    return scores, deltas


def _decode_clip(a, d, img_hw):
    cx = d[:, 0] * a[:, 2] + a[:, 0]
    cy = d[:, 1] * a[:, 3] + a[:, 1]
    w = a[:, 2] * jnp.exp(jnp.clip(d[:, 2], -4.0, 4.0))
    h = a[:, 3] * jnp.exp(jnp.clip(d[:, 3], -4.0, 4.0))
    hh = img_hw[0].astype(jnp.float32)
    ww = img_hw[1].astype(jnp.float32)
    return jnp.stack([
        jnp.clip(cx - 0.5 * w, 0.0, ww), jnp.clip(cy - 0.5 * h, 0.0, hh),
        jnp.clip(cx + 0.5 * w, 0.0, ww), jnp.clip(cy + 0.5 * h, 0.0, hh)], axis=-1)


def _one_image(score, delta, img_hw):
    topv, topi = lax.top_k(score, PRE_K)
    boxes = _decode_clip(_ANCH[topi], delta[topi], img_hw)
    area = jnp.maximum(boxes[:, 2] - boxes[:, 0], 0.0) * jnp.maximum(boxes[:, 3] - boxes[:, 1], 0.0)
    ix1 = jnp.maximum(boxes[:, None, 0], boxes[None, :, 0])
    iy1 = jnp.maximum(boxes[:, None, 1], boxes[None, :, 1])
    ix2 = jnp.minimum(boxes[:, None, 2], boxes[None, :, 2])
    iy2 = jnp.minimum(boxes[:, None, 3], boxes[None, :, 3])
    inter = jnp.maximum(ix2 - ix1, 0.0) * jnp.maximum(iy2 - iy1, 0.0)
    iou = inter / jnp.maximum(area[:, None] + area[None, :] - inter, 1e-6)
    idx = jnp.arange(PRE_K)

    def body(i, keep):
        sup = (iou[i] > NMS_THR) & (idx > i) & keep[i]
        return keep & (~sup)

    keep = lax.fori_loop(0, PRE_K, body, jnp.ones((PRE_K,), bool))
    masked = jnp.where(keep, topv, -1.0)
    sv, si = lax.top_k(masked, POST_K)
    return boxes[si], sv


def kernel(features, img_size, conv_w, conv_b, cls_w, cls_b, reg_w, reg_b):
    scores, deltas = _rpn_head(features, conv_w, conv_b, cls_w, cls_b, reg_w, reg_b)
    proposals, pscores = jax.vmap(_one_image)(scores, deltas, img_size)
    return proposals, pscores
